# baseline (speedup 1.0000x reference)
import functools

import jax
import jax.numpy as jnp
from jax import lax
from jax.experimental import pallas as pl
from jax.experimental.pallas import tpu as pltpu
from jax.experimental.pallas import tpu_sc as plsc

N = 10000
E = 320000
D = 128
G = 10

NC = 2
NS = 16
C = 80
NCHUNKS = E // C
CPC = NCHUNKS // NC
CPT = CPC // NS
MAIN = 124
NBUF = 4
IGRP = 4
NGRP = MAIN // IGRP
SLOTS = 3
IROWS = SLOTS * IGRP
WB = 624
WREM = N - NS * WB
ZR = 48

def _sc_agg_body(h_hbm, src_hbm, dst_hbm, out_hbm, sidx, didx, rows, zbuf, acc,
                 isem, gsems, ssems):
    c = lax.axis_index("c")
    s = lax.axis_index("s")

    zv = jnp.zeros((16,), jnp.float32)

    def _zrow(r, carry):
        for j in range(D // 16):
            zbuf[r, pl.ds(j * 16, 16)] = zv
        return carry

    lax.fori_loop(0, ZR, _zrow, 0)

    for k in range(WB // ZR):
        pltpu.sync_copy(zbuf, acc.at[pl.ds(s * WB + k * ZR, ZR)])

    @pl.when(s == NS - 1)
    def _zrem():
        pltpu.sync_copy(zbuf.at[pl.ds(0, WREM)], acc.at[pl.ds(NS * WB, WREM)])

    plsc.subcore_barrier()

    base = c * CPC + s * CPT

    def _idx_load(grp, slot):
        off = pl.ds(slot * IGRP, IGRP)
        pltpu.async_copy(src_hbm.at[pl.ds(base + grp * IGRP, IGRP)],
                         sidx.at[off], isem)
        pltpu.async_copy(dst_hbm.at[pl.ds(base + grp * IGRP, IGRP)],
                         didx.at[off], isem)

    def _idx_wait():
        for _ in range(2):
            pltpu.make_async_copy(src_hbm.at[pl.ds(0, IGRP)],
                                  sidx.at[pl.ds(0, IGRP)], isem).wait()

    def _irow(j):
        return lax.rem(j // IGRP, SLOTS) * IGRP + lax.rem(j, IGRP)

    def _gather(j, b):
        pltpu.async_copy(h_hbm.at[sidx.at[_irow(j), 0]], rows.at[b], gsems[b])

    def _gather_wait(b):
        pltpu.make_async_copy(h_hbm.at[pl.ds(0, C)], rows.at[b],
                              gsems[b]).wait()

    def _scatter(j, b):
        pltpu.async_copy(rows.at[b], acc.at[didx.at[_irow(j), 0]],
                         ssems[b], add=True)

    def _swait(b):
        pltpu.make_async_copy(rows.at[b], acc.at[pl.ds(0, C)],
                              ssems[b]).wait()

    _idx_load(0, 0)
    _idx_wait()
    _idx_load(1, 1)
    _gather(0, 0)
    _gather(1, 1)

    _idx_wait()
    for k in range(IGRP):
        _gather_wait(k)
        _scatter(k, k)
        if k >= 2:
            _swait((k + 2) % NBUF)
        if k == 2:
            _idx_load(2, 2)
        _gather(k + 2, (k + 2) % NBUF)

    def _outer(g, carry):
        @pl.when(g + 1 < NGRP)
        def _():
            _idx_wait()

        for k in range(IGRP):
            i = g * IGRP + k
            _gather_wait(k)
            _scatter(i, k)
            _swait((k + 2) % NBUF)

            if k == 2:
                @pl.when(g + 2 < NGRP)
                def _():
                    _idx_load(g + 2, lax.rem(g - 1, SLOTS))

            gn = i + 2

            @pl.when(gn < MAIN)
            def _():
                _gather(gn, (k + 2) % NBUF)

        return carry

    lax.fori_loop(1, NGRP, _outer, 0)

    _swait((MAIN - 2) % NBUF)
    _swait((MAIN - 1) % NBUF)

    pltpu.sync_copy(src_hbm.at[base + MAIN], sidx.at[0])
    pltpu.sync_copy(dst_hbm.at[base + MAIN], didx.at[0])
    pltpu.async_copy(h_hbm.at[sidx.at[0, 0]], rows.at[0], gsems[0]).wait()
    pltpu.sync_copy(rows.at[0], acc.at[didx.at[0, 0]], add=True)

    plsc.subcore_barrier()

    pltpu.sync_copy(acc.at[pl.ds(s * WB, WB)],
                    out_hbm.at[c, pl.ds(s * WB, WB)])

    @pl.when(s == NS - 1)
    def _wrem():
        pltpu.sync_copy(acc.at[pl.ds(NS * WB, WREM)],
                        out_hbm.at[c, pl.ds(NS * WB, WREM)])


@functools.cache
def _get_sc_agg():
    mesh = plsc.VectorSubcoreMesh(core_axis_name="c", subcore_axis_name="s",
                                  num_cores=NC, num_subcores=NS)
    return pl.kernel(
        _sc_agg_body,
        out_type=jax.ShapeDtypeStruct((NC, N, D), jnp.float32),
        mesh=mesh,
        scratch_types=[
            pltpu.VMEM((IROWS, 1, C), jnp.int32),
            pltpu.VMEM((IROWS, 1, C), jnp.int32),
            pltpu.VMEM((NBUF, C, D), jnp.float32),
            pltpu.VMEM((ZR, D), jnp.float32),
            pltpu.VMEM_SHARED((N, D), jnp.float32),
            pltpu.SemaphoreType.DMA,
            [pltpu.SemaphoreType.DMA] * NBUF,
            [pltpu.SemaphoreType.DMA] * NBUF,
        ],
    )


RB = 2000
PB = N // G


def _mid_body(x_ref, a_ref, w1_ref, b1_ref, w2_ref, b2_ref, o_ref):
    z = x_ref[...] + a_ref[0] + a_ref[1]
    z = jnp.dot(z, w1_ref[...], preferred_element_type=jnp.float32) + b1_ref[...]
    z = jnp.maximum(z, 0.0)
    z = jnp.dot(z, w2_ref[...], preferred_element_type=jnp.float32) + b2_ref[...]
    o_ref[...] = jnp.maximum(z, 0.0)


def _mlp_mid(h, agg, W1, b1, W2, b2):
    return pl.pallas_call(
        _mid_body,
        grid=(N // RB,),
        in_specs=[
            pl.BlockSpec((RB, D), lambda g: (g, 0)),
            pl.BlockSpec((NC, RB, D), lambda g: (0, g, 0)),
            pl.BlockSpec((D, D), lambda g: (0, 0)),
            pl.BlockSpec((1, D), lambda g: (0, 0)),
            pl.BlockSpec((D, D), lambda g: (0, 0)),
            pl.BlockSpec((1, D), lambda g: (0, 0)),
        ],
        out_specs=pl.BlockSpec((RB, D), lambda g: (g, 0)),
        out_shape=jax.ShapeDtypeStruct((N, D), jnp.float32),
    )(h, agg, W1, b1.reshape(1, D), W2, b2.reshape(1, D))


def _pool_body(x_ref, a_ref, w1_ref, b1_ref, w2_ref, b2_ref, o_ref):
    z = x_ref[...] + a_ref[0] + a_ref[1]
    z = jnp.dot(z, w1_ref[...], preferred_element_type=jnp.float32) + b1_ref[...]
    z = jnp.maximum(z, 0.0)
    y = jnp.dot(z, w2_ref[...], preferred_element_type=jnp.float32) + b2_ref[...]
    o_ref[0] = jnp.sum(y, axis=0, keepdims=True) * (1.0 / PB)


def _mlp_pool(h, agg, W1, b1, W2, b2):
    return pl.pallas_call(
        _pool_body,
        grid=(G,),
        in_specs=[
            pl.BlockSpec((PB, D), lambda g: (g, 0)),
            pl.BlockSpec((NC, PB, D), lambda g: (0, g, 0)),
            pl.BlockSpec((D, D), lambda g: (0, 0)),
            pl.BlockSpec((1, D), lambda g: (0, 0)),
            pl.BlockSpec((D, D), lambda g: (0, 0)),
            pl.BlockSpec((1, D), lambda g: (0, 0)),
        ],
        out_specs=pl.BlockSpec((1, 1, D), lambda g: (g, 0, 0)),
        out_shape=jax.ShapeDtypeStruct((G, 1, D), jnp.float32),
    )(h, agg, W1, b1.reshape(1, D), W2, b2.reshape(1, D)).reshape(G, D)


def kernel(x, edge_index, ptr, W1_0, b1_0, W2_0, b2_0, W1_1, b1_1, W2_1, b2_1):
    src = edge_index[0].reshape(NCHUNKS, 1, C)
    dst = edge_index[1].reshape(NCHUNKS, 1, C)
    sc_agg = _get_sc_agg()
    agg0 = sc_agg(x, src, dst)
    h1 = _mlp_mid(x, agg0, W1_0, b1_0, W2_0, b2_0)
    agg1 = sc_agg(h1, src, dst)
    return _mlp_pool(h1, agg1, W1_1, b1_1, W2_1, b2_1)


# speedup vs baseline: 1.1115x; 1.1115x over previous
import functools

import jax
import jax.numpy as jnp
from jax import lax
from jax.experimental import pallas as pl
from jax.experimental.pallas import tpu as pltpu
from jax.experimental.pallas import tpu_sc as plsc

N = 10000
E = 320000
D = 128
G = 10

NC = 2
NS = 16
C = 100
NCHUNKS = E // C
CPC = NCHUNKS // NC
CPT = CPC // NS
MAIN = 99
NBUF = 3
IGRP = 3
NGRP = MAIN // IGRP
SLOTS = 3
IROWS = SLOTS * IGRP
WB = 624
WREM = N - NS * WB
ZR = 48

def _sc_agg_body(h_hbm, src_hbm, dst_hbm, out_hbm, sidx, didx, rows, zbuf, acc,
                 isem, gsems, ssems):
    c = lax.axis_index("c")
    s = lax.axis_index("s")

    zv = jnp.zeros((16,), jnp.float32)

    def _zrow(r, carry):
        for j in range(D // 16):
            zbuf[r, pl.ds(j * 16, 16)] = zv
        return carry

    lax.fori_loop(0, ZR, _zrow, 0)

    for k in range(WB // ZR):
        pltpu.sync_copy(zbuf, acc.at[pl.ds(s * WB + k * ZR, ZR)])

    @pl.when(s == NS - 1)
    def _zrem():
        pltpu.sync_copy(zbuf.at[pl.ds(0, WREM)], acc.at[pl.ds(NS * WB, WREM)])

    plsc.subcore_barrier()

    base = c * CPC + s * CPT

    def _idx_load(grp, slot):
        off = pl.ds(slot * IGRP, IGRP)
        pltpu.async_copy(src_hbm.at[pl.ds(base + grp * IGRP, IGRP)],
                         sidx.at[off], isem)
        pltpu.async_copy(dst_hbm.at[pl.ds(base + grp * IGRP, IGRP)],
                         didx.at[off], isem)

    def _idx_wait():
        for _ in range(2):
            pltpu.make_async_copy(src_hbm.at[pl.ds(0, IGRP)],
                                  sidx.at[pl.ds(0, IGRP)], isem).wait()

    def _irow(j):
        return lax.rem(j // IGRP, SLOTS) * IGRP + lax.rem(j, IGRP)

    def _gather(j, b):
        pltpu.async_copy(h_hbm.at[sidx.at[_irow(j), 0]], rows.at[b], gsems[b])

    def _gather_wait(b):
        pltpu.make_async_copy(h_hbm.at[sidx.at[0, 0]], rows.at[b],
                              gsems[b]).wait()

    def _scatter(j, b):
        pltpu.async_copy(rows.at[b], acc.at[didx.at[_irow(j), 0]],
                         ssems[b], add=True)

    def _swait(b):
        pltpu.make_async_copy(rows.at[b], acc.at[didx.at[0, 0]],
                              ssems[b]).wait()

    _idx_load(0, 0)
    _idx_wait()
    _idx_load(1, 1)
    _gather(0, 0)
    _gather(1, 1)

    _idx_wait()
    for k in range(IGRP):
        _gather_wait(k)
        _scatter(k, k)
        if k >= 1:
            _swait((k + 2) % NBUF)
        if k == 2:
            _idx_load(2, 2)
        _gather(k + 2, (k + 2) % NBUF)

    def _outer(g, carry):
        @pl.when(g + 1 < NGRP)
        def _():
            _idx_wait()

        for k in range(IGRP):
            i = g * IGRP + k
            _gather_wait(k)
            _scatter(i, k)
            _swait((k + 2) % NBUF)

            if k == 2:
                @pl.when(g + 2 < NGRP)
                def _():
                    _idx_load(g + 2, lax.rem(g - 1, SLOTS))

            gn = i + 2

            @pl.when(gn < MAIN)
            def _():
                _gather(gn, (k + 2) % NBUF)

        return carry

    lax.fori_loop(1, NGRP, _outer, 0)

    _swait((MAIN - 1) % NBUF)

    pltpu.sync_copy(src_hbm.at[base + MAIN], sidx.at[0])
    pltpu.sync_copy(dst_hbm.at[base + MAIN], didx.at[0])
    pltpu.async_copy(h_hbm.at[sidx.at[0, 0]], rows.at[0], gsems[0]).wait()
    pltpu.sync_copy(rows.at[0], acc.at[didx.at[0, 0]], add=True)

    plsc.subcore_barrier()

    pltpu.sync_copy(acc.at[pl.ds(s * WB, WB)],
                    out_hbm.at[c, pl.ds(s * WB, WB)])

    @pl.when(s == NS - 1)
    def _wrem():
        pltpu.sync_copy(acc.at[pl.ds(NS * WB, WREM)],
                        out_hbm.at[c, pl.ds(NS * WB, WREM)])


@functools.cache
def _get_sc_agg():
    mesh = plsc.VectorSubcoreMesh(core_axis_name="c", subcore_axis_name="s",
                                  num_cores=NC, num_subcores=NS)
    return pl.kernel(
        _sc_agg_body,
        out_type=jax.ShapeDtypeStruct((NC, N, D), jnp.float32),
        mesh=mesh,
        scratch_types=[
            pltpu.VMEM((IROWS, 1, C), jnp.int32),
            pltpu.VMEM((IROWS, 1, C), jnp.int32),
            pltpu.VMEM((NBUF, C, D), jnp.float32),
            pltpu.VMEM((ZR, D), jnp.float32),
            pltpu.VMEM_SHARED((N, D), jnp.float32),
            pltpu.SemaphoreType.DMA,
            [pltpu.SemaphoreType.DMA] * NBUF,
            [pltpu.SemaphoreType.DMA] * NBUF,
        ],
    )


RB = 2000
PB = N // G


def _mid_body(x_ref, a_ref, w1_ref, b1_ref, w2_ref, b2_ref, o_ref):
    z = x_ref[...] + a_ref[0] + a_ref[1]
    z = jnp.dot(z, w1_ref[...], preferred_element_type=jnp.float32) + b1_ref[...]
    z = jnp.maximum(z, 0.0)
    z = jnp.dot(z, w2_ref[...], preferred_element_type=jnp.float32) + b2_ref[...]
    o_ref[...] = jnp.maximum(z, 0.0)


def _mlp_mid(h, agg, W1, b1, W2, b2):
    return pl.pallas_call(
        _mid_body,
        grid=(N // RB,),
        in_specs=[
            pl.BlockSpec((RB, D), lambda g: (g, 0)),
            pl.BlockSpec((NC, RB, D), lambda g: (0, g, 0)),
            pl.BlockSpec((D, D), lambda g: (0, 0)),
            pl.BlockSpec((1, D), lambda g: (0, 0)),
            pl.BlockSpec((D, D), lambda g: (0, 0)),
            pl.BlockSpec((1, D), lambda g: (0, 0)),
        ],
        out_specs=pl.BlockSpec((RB, D), lambda g: (g, 0)),
        out_shape=jax.ShapeDtypeStruct((N, D), jnp.float32),
    )(h, agg, W1, b1.reshape(1, D), W2, b2.reshape(1, D))


def _pool_body(x_ref, a_ref, w1_ref, b1_ref, w2_ref, b2_ref, o_ref):
    z = x_ref[...] + a_ref[0] + a_ref[1]
    z = jnp.dot(z, w1_ref[...], preferred_element_type=jnp.float32) + b1_ref[...]
    z = jnp.maximum(z, 0.0)
    y = jnp.dot(z, w2_ref[...], preferred_element_type=jnp.float32) + b2_ref[...]
    o_ref[0] = jnp.sum(y, axis=0, keepdims=True) * (1.0 / PB)


def _mlp_pool(h, agg, W1, b1, W2, b2):
    return pl.pallas_call(
        _pool_body,
        grid=(G,),
        in_specs=[
            pl.BlockSpec((PB, D), lambda g: (g, 0)),
            pl.BlockSpec((NC, PB, D), lambda g: (0, g, 0)),
            pl.BlockSpec((D, D), lambda g: (0, 0)),
            pl.BlockSpec((1, D), lambda g: (0, 0)),
            pl.BlockSpec((D, D), lambda g: (0, 0)),
            pl.BlockSpec((1, D), lambda g: (0, 0)),
        ],
        out_specs=pl.BlockSpec((1, 1, D), lambda g: (g, 0, 0)),
        out_shape=jax.ShapeDtypeStruct((G, 1, D), jnp.float32),
    )(h, agg, W1, b1.reshape(1, D), W2, b2.reshape(1, D)).reshape(G, D)


def kernel(x, edge_index, ptr, W1_0, b1_0, W2_0, b2_0, W1_1, b1_1, W2_1, b2_1):
    src = edge_index[0].reshape(NCHUNKS, 1, C)
    dst = edge_index[1].reshape(NCHUNKS, 1, C)
    sc_agg = _get_sc_agg()
    agg0 = sc_agg(x, src, dst)
    h1 = _mlp_mid(x, agg0, W1_0, b1_0, W2_0, b2_0)
    agg1 = sc_agg(h1, src, dst)
    return _mlp_pool(h1, agg1, W1_1, b1_1, W2_1, b2_1)


# speedup vs baseline: 1.1793x; 1.0609x over previous
import functools

import jax
import jax.numpy as jnp
from jax import lax
from jax.experimental import pallas as pl
from jax.experimental.pallas import tpu as pltpu
from jax.experimental.pallas import tpu_sc as plsc

N = 10000
E = 320000
D = 128
G = 10

NC = 2
NS = 16
C = 128
NCHUNKS = E // C
CPC = NCHUNKS // NC
MAIN = 78
NREM = CPC - NS * MAIN
NBUF = 3
IGRP = 3
NGRP = MAIN // IGRP
SLOTS = 3
IROWS = SLOTS * IGRP
WB = 624
WREM = N - NS * WB
ZR = 48

def _sc_agg_body(h_hbm, src_hbm, dst_hbm, out_hbm, sidx, didx, rows, zbuf, acc,
                 isem, gsems, ssems):
    c = lax.axis_index("c")
    s = lax.axis_index("s")

    zv = jnp.zeros((32,), jnp.bfloat16)

    def _zrow(r, carry):
        for j in range(D // 32):
            zbuf[r, pl.ds(j * 32, 32)] = zv
        return carry

    lax.fori_loop(0, ZR, _zrow, 0)

    for k in range(WB // ZR):
        pltpu.sync_copy(zbuf, acc.at[pl.ds(s * WB + k * ZR, ZR)])

    @pl.when(s == NS - 1)
    def _zrem():
        pltpu.sync_copy(zbuf.at[pl.ds(0, WREM)], acc.at[pl.ds(NS * WB, WREM)])

    plsc.subcore_barrier()

    base = c * CPC + s * MAIN

    def _idx_load(grp, slot):
        off = pl.ds(slot * IGRP, IGRP)
        pltpu.async_copy(src_hbm.at[pl.ds(base + grp * IGRP, IGRP)],
                         sidx.at[off], isem)
        pltpu.async_copy(dst_hbm.at[pl.ds(base + grp * IGRP, IGRP)],
                         didx.at[off], isem)

    def _idx_wait():
        for _ in range(2):
            pltpu.make_async_copy(src_hbm.at[pl.ds(0, IGRP)],
                                  sidx.at[pl.ds(0, IGRP)], isem).wait()

    def _irow(j):
        return lax.rem(j // IGRP, SLOTS) * IGRP + lax.rem(j, IGRP)

    def _gather(j, b):
        pltpu.async_copy(h_hbm.at[sidx.at[_irow(j), 0]], rows.at[b], gsems[b])

    def _gather_wait(b):
        pltpu.make_async_copy(h_hbm.at[sidx.at[0, 0]], rows.at[b],
                              gsems[b]).wait()

    def _scatter(j, b):
        pltpu.async_copy(rows.at[b], acc.at[didx.at[_irow(j), 0]],
                         ssems[b], add=True)

    def _swait(b):
        pltpu.make_async_copy(rows.at[b], acc.at[didx.at[0, 0]],
                              ssems[b]).wait()

    _idx_load(0, 0)
    _idx_wait()
    _idx_load(1, 1)
    _gather(0, 0)
    _gather(1, 1)

    _idx_wait()
    for k in range(IGRP):
        _gather_wait(k)
        _scatter(k, k)
        if k >= 1:
            _swait((k + 2) % NBUF)
        if k == 2:
            _idx_load(2, 2)
        _gather(k + 2, (k + 2) % NBUF)

    def _outer(g, carry):
        @pl.when(g + 1 < NGRP)
        def _():
            _idx_wait()

        for k in range(IGRP):
            i = g * IGRP + k
            _gather_wait(k)
            _scatter(i, k)
            _swait((k + 2) % NBUF)

            if k == 2:
                @pl.when(g + 2 < NGRP)
                def _():
                    _idx_load(g + 2, lax.rem(g - 1, SLOTS))

            gn = i + 2

            @pl.when(gn < MAIN)
            def _():
                _gather(gn, (k + 2) % NBUF)

        return carry

    lax.fori_loop(1, NGRP, _outer, 0)

    _swait((MAIN - 1) % NBUF)

    @pl.when(s < NREM)
    def _rem():
        chunk = c * CPC + NS * MAIN + s
        pltpu.sync_copy(src_hbm.at[chunk], sidx.at[0])
        pltpu.sync_copy(dst_hbm.at[chunk], didx.at[0])
        pltpu.async_copy(h_hbm.at[sidx.at[0, 0]], rows.at[0], gsems[0]).wait()
        pltpu.sync_copy(rows.at[0], acc.at[didx.at[0, 0]], add=True)

    plsc.subcore_barrier()

    pltpu.sync_copy(acc.at[pl.ds(s * WB, WB)],
                    out_hbm.at[c, pl.ds(s * WB, WB)])

    @pl.when(s == NS - 1)
    def _wrem():
        pltpu.sync_copy(acc.at[pl.ds(NS * WB, WREM)],
                        out_hbm.at[c, pl.ds(NS * WB, WREM)])


@functools.cache
def _get_sc_agg():
    mesh = plsc.VectorSubcoreMesh(core_axis_name="c", subcore_axis_name="s",
                                  num_cores=NC, num_subcores=NS)
    return pl.kernel(
        _sc_agg_body,
        out_type=jax.ShapeDtypeStruct((NC, N, D), jnp.bfloat16),
        mesh=mesh,
        compiler_params=pltpu.CompilerParams(use_tc_tiling_on_sc=False),
        scratch_types=[
            pltpu.VMEM((IROWS, 1, C), jnp.int32),
            pltpu.VMEM((IROWS, 1, C), jnp.int32),
            pltpu.VMEM((NBUF, C, D), jnp.bfloat16),
            pltpu.VMEM((ZR, D), jnp.bfloat16),
            pltpu.VMEM_SHARED((N, D), jnp.bfloat16),
            pltpu.SemaphoreType.DMA,
            [pltpu.SemaphoreType.DMA] * NBUF,
            [pltpu.SemaphoreType.DMA] * NBUF,
        ],
    )


RB = 2000
PB = N // G


def _cast_body(x_ref, o_ref):
    o_ref[...] = x_ref[...].astype(jnp.bfloat16)


def _to_bf16(x):
    return pl.pallas_call(
        _cast_body,
        grid=(N // RB,),
        in_specs=[pl.BlockSpec((RB, D), lambda g: (g, 0))],
        out_specs=pl.BlockSpec((RB, D), lambda g: (g, 0)),
        out_shape=jax.ShapeDtypeStruct((N, D), jnp.bfloat16),
    )(x)


def _mid_body(x_ref, a_ref, w1_ref, b1_ref, w2_ref, b2_ref, o_ref):
    z = (x_ref[...] + a_ref[0].astype(jnp.float32)
         + a_ref[1].astype(jnp.float32))
    z = jnp.dot(z, w1_ref[...], preferred_element_type=jnp.float32) + b1_ref[...]
    z = jnp.maximum(z, 0.0)
    z = jnp.dot(z, w2_ref[...], preferred_element_type=jnp.float32) + b2_ref[...]
    o_ref[...] = jnp.maximum(z, 0.0).astype(jnp.bfloat16)


def _mlp_mid(h, agg, W1, b1, W2, b2):
    return pl.pallas_call(
        _mid_body,
        grid=(N // RB,),
        in_specs=[
            pl.BlockSpec((RB, D), lambda g: (g, 0)),
            pl.BlockSpec((NC, RB, D), lambda g: (0, g, 0)),
            pl.BlockSpec((D, D), lambda g: (0, 0)),
            pl.BlockSpec((1, D), lambda g: (0, 0)),
            pl.BlockSpec((D, D), lambda g: (0, 0)),
            pl.BlockSpec((1, D), lambda g: (0, 0)),
        ],
        out_specs=pl.BlockSpec((RB, D), lambda g: (g, 0)),
        out_shape=jax.ShapeDtypeStruct((N, D), jnp.bfloat16),
    )(h, agg, W1, b1.reshape(1, D), W2, b2.reshape(1, D))


def _pool_body(x_ref, a_ref, w1_ref, b1_ref, w2_ref, b2_ref, o_ref):
    z = (x_ref[...].astype(jnp.float32) + a_ref[0].astype(jnp.float32)
         + a_ref[1].astype(jnp.float32))
    z = jnp.dot(z, w1_ref[...], preferred_element_type=jnp.float32) + b1_ref[...]
    z = jnp.maximum(z, 0.0)
    y = jnp.dot(z, w2_ref[...], preferred_element_type=jnp.float32) + b2_ref[...]
    o_ref[0] = jnp.sum(y, axis=0, keepdims=True) * (1.0 / PB)


def _mlp_pool(h, agg, W1, b1, W2, b2):
    return pl.pallas_call(
        _pool_body,
        grid=(G,),
        in_specs=[
            pl.BlockSpec((PB, D), lambda g: (g, 0)),
            pl.BlockSpec((NC, PB, D), lambda g: (0, g, 0)),
            pl.BlockSpec((D, D), lambda g: (0, 0)),
            pl.BlockSpec((1, D), lambda g: (0, 0)),
            pl.BlockSpec((D, D), lambda g: (0, 0)),
            pl.BlockSpec((1, D), lambda g: (0, 0)),
        ],
        out_specs=pl.BlockSpec((1, 1, D), lambda g: (g, 0, 0)),
        out_shape=jax.ShapeDtypeStruct((G, 1, D), jnp.float32),
    )(h, agg, W1, b1.reshape(1, D), W2, b2.reshape(1, D)).reshape(G, D)


def kernel(x, edge_index, ptr, W1_0, b1_0, W2_0, b2_0, W1_1, b1_1, W2_1, b2_1):
    src = edge_index[0].reshape(NCHUNKS, 1, C)
    dst = edge_index[1].reshape(NCHUNKS, 1, C)
    sc_agg = _get_sc_agg()
    xb = _to_bf16(x)
    agg0 = sc_agg(xb, src, dst)
    h1 = _mlp_mid(x, agg0, W1_0, b1_0, W2_0, b2_0)
    agg1 = sc_agg(h1, src, dst)
    return _mlp_pool(h1, agg1, W1_1, b1_1, W2_1, b2_1)


# speedup vs baseline: 1.2037x; 1.0207x over previous
import functools

import jax
import jax.numpy as jnp
from jax import lax
from jax.experimental import pallas as pl
from jax.experimental.pallas import tpu as pltpu
from jax.experimental.pallas import tpu_sc as plsc

N = 10000
E = 320000
D = 128
G = 10

NC = 2
NS = 16
C = 128
NCHUNKS = E // C
CPC = NCHUNKS // NC
MAIN = 78
NREM = CPC - NS * MAIN
NBUF = 3
IGRP = 6
NGRP = MAIN // IGRP
SLOTS = 3
IROWS = SLOTS * IGRP
WB = 624
WREM = N - NS * WB
ZR = 48

def _sc_agg_body(h_hbm, src_hbm, dst_hbm, out_hbm, sidx, didx, rows, zbuf, acc,
                 isem, gsems, ssems):
    c = lax.axis_index("c")
    s = lax.axis_index("s")

    zv = jnp.zeros((32,), jnp.bfloat16)

    def _zrow(r, carry):
        for j in range(D // 32):
            zbuf[r, pl.ds(j * 32, 32)] = zv
        return carry

    lax.fori_loop(0, ZR, _zrow, 0)

    for k in range(WB // ZR):
        pltpu.sync_copy(zbuf, acc.at[pl.ds(s * WB + k * ZR, ZR)])

    @pl.when(s == NS - 1)
    def _zrem():
        pltpu.sync_copy(zbuf.at[pl.ds(0, WREM)], acc.at[pl.ds(NS * WB, WREM)])

    plsc.subcore_barrier()

    base = c * CPC + s * MAIN

    def _idx_load(grp, slot):
        off = pl.ds(slot * IGRP, IGRP)
        pltpu.async_copy(src_hbm.at[pl.ds(base + grp * IGRP, IGRP)],
                         sidx.at[off], isem)
        pltpu.async_copy(dst_hbm.at[pl.ds(base + grp * IGRP, IGRP)],
                         didx.at[off], isem)

    def _idx_wait():
        for _ in range(2):
            pltpu.make_async_copy(src_hbm.at[pl.ds(0, IGRP)],
                                  sidx.at[pl.ds(0, IGRP)], isem).wait()

    def _irow(j):
        return lax.rem(j // IGRP, SLOTS) * IGRP + lax.rem(j, IGRP)

    def _gather(j, b):
        pltpu.async_copy(h_hbm.at[sidx.at[_irow(j), 0]], rows.at[b], gsems[b])

    def _gather_wait(b):
        pltpu.make_async_copy(h_hbm.at[sidx.at[0, 0]], rows.at[b],
                              gsems[b]).wait()

    def _scatter(j, b):
        pltpu.async_copy(rows.at[b], acc.at[didx.at[_irow(j), 0]],
                         ssems[b], add=True)

    def _swait(b):
        pltpu.make_async_copy(rows.at[b], acc.at[didx.at[0, 0]],
                              ssems[b]).wait()

    _idx_load(0, 0)
    _idx_wait()
    _idx_load(1, 1)
    _gather(0, 0)
    _gather(1, 1)

    _idx_wait()
    for k in range(IGRP):
        b = k % NBUF
        _gather_wait(b)
        _scatter(k, b)
        if k >= 1:
            _swait((b + 2) % NBUF)
        if k == 2:
            _idx_load(2, 2)
        _gather(k + 2, (b + 2) % NBUF)

    def _outer(g, carry):
        @pl.when(g + 1 < NGRP)
        def _():
            _idx_wait()

        for k in range(IGRP):
            i = g * IGRP + k
            b = k % NBUF
            _gather_wait(b)
            _scatter(i, b)
            _swait((b + 2) % NBUF)

            if k == 2:
                @pl.when(g + 2 < NGRP)
                def _():
                    _idx_load(g + 2, lax.rem(g - 1, SLOTS))

            gn = i + 2

            @pl.when(gn < MAIN)
            def _():
                _gather(gn, (b + 2) % NBUF)

        return carry

    lax.fori_loop(1, NGRP, _outer, 0)

    _swait((MAIN - 1) % NBUF)

    @pl.when(s < NREM)
    def _rem():
        chunk = c * CPC + NS * MAIN + s
        pltpu.sync_copy(src_hbm.at[chunk], sidx.at[0])
        pltpu.sync_copy(dst_hbm.at[chunk], didx.at[0])
        pltpu.async_copy(h_hbm.at[sidx.at[0, 0]], rows.at[0], gsems[0]).wait()
        pltpu.sync_copy(rows.at[0], acc.at[didx.at[0, 0]], add=True)

    plsc.subcore_barrier()

    pltpu.sync_copy(acc.at[pl.ds(s * WB, WB)],
                    out_hbm.at[c, pl.ds(s * WB, WB)])

    @pl.when(s == NS - 1)
    def _wrem():
        pltpu.sync_copy(acc.at[pl.ds(NS * WB, WREM)],
                        out_hbm.at[c, pl.ds(NS * WB, WREM)])


@functools.cache
def _get_sc_agg():
    mesh = plsc.VectorSubcoreMesh(core_axis_name="c", subcore_axis_name="s",
                                  num_cores=NC, num_subcores=NS)
    return pl.kernel(
        _sc_agg_body,
        out_type=jax.ShapeDtypeStruct((NC, N, D), jnp.bfloat16),
        mesh=mesh,
        compiler_params=pltpu.CompilerParams(use_tc_tiling_on_sc=False),
        scratch_types=[
            pltpu.VMEM((IROWS, 1, C), jnp.int32),
            pltpu.VMEM((IROWS, 1, C), jnp.int32),
            pltpu.VMEM((NBUF, C, D), jnp.bfloat16),
            pltpu.VMEM((ZR, D), jnp.bfloat16),
            pltpu.VMEM_SHARED((N, D), jnp.bfloat16),
            pltpu.SemaphoreType.DMA,
            [pltpu.SemaphoreType.DMA] * NBUF,
            [pltpu.SemaphoreType.DMA] * NBUF,
        ],
    )


RB = 2000
PB = N // G


def _cast_body(x_ref, o_ref):
    o_ref[...] = x_ref[...].astype(jnp.bfloat16)


def _to_bf16(x):
    return pl.pallas_call(
        _cast_body,
        grid=(N // RB,),
        in_specs=[pl.BlockSpec((RB, D), lambda g: (g, 0))],
        out_specs=pl.BlockSpec((RB, D), lambda g: (g, 0)),
        out_shape=jax.ShapeDtypeStruct((N, D), jnp.bfloat16),
    )(x)


def _mid_body(x_ref, a_ref, w1_ref, b1_ref, w2_ref, b2_ref, o_ref):
    z = (x_ref[...] + a_ref[0].astype(jnp.float32)
         + a_ref[1].astype(jnp.float32))
    z = jnp.dot(z, w1_ref[...], preferred_element_type=jnp.float32) + b1_ref[...]
    z = jnp.maximum(z, 0.0)
    z = jnp.dot(z, w2_ref[...], preferred_element_type=jnp.float32) + b2_ref[...]
    o_ref[...] = jnp.maximum(z, 0.0).astype(jnp.bfloat16)


def _mlp_mid(h, agg, W1, b1, W2, b2):
    return pl.pallas_call(
        _mid_body,
        grid=(N // RB,),
        in_specs=[
            pl.BlockSpec((RB, D), lambda g: (g, 0)),
            pl.BlockSpec((NC, RB, D), lambda g: (0, g, 0)),
            pl.BlockSpec((D, D), lambda g: (0, 0)),
            pl.BlockSpec((1, D), lambda g: (0, 0)),
            pl.BlockSpec((D, D), lambda g: (0, 0)),
            pl.BlockSpec((1, D), lambda g: (0, 0)),
        ],
        out_specs=pl.BlockSpec((RB, D), lambda g: (g, 0)),
        out_shape=jax.ShapeDtypeStruct((N, D), jnp.bfloat16),
    )(h, agg, W1, b1.reshape(1, D), W2, b2.reshape(1, D))


def _pool_body(x_ref, a_ref, w1_ref, b1_ref, w2_ref, b2_ref, o_ref):
    z = (x_ref[...].astype(jnp.float32) + a_ref[0].astype(jnp.float32)
         + a_ref[1].astype(jnp.float32))
    z = jnp.dot(z, w1_ref[...], preferred_element_type=jnp.float32) + b1_ref[...]
    z = jnp.maximum(z, 0.0)
    y = jnp.dot(z, w2_ref[...], preferred_element_type=jnp.float32) + b2_ref[...]
    o_ref[0] = jnp.sum(y, axis=0, keepdims=True) * (1.0 / PB)


def _mlp_pool(h, agg, W1, b1, W2, b2):
    return pl.pallas_call(
        _pool_body,
        grid=(G,),
        in_specs=[
            pl.BlockSpec((PB, D), lambda g: (g, 0)),
            pl.BlockSpec((NC, PB, D), lambda g: (0, g, 0)),
            pl.BlockSpec((D, D), lambda g: (0, 0)),
            pl.BlockSpec((1, D), lambda g: (0, 0)),
            pl.BlockSpec((D, D), lambda g: (0, 0)),
            pl.BlockSpec((1, D), lambda g: (0, 0)),
        ],
        out_specs=pl.BlockSpec((1, 1, D), lambda g: (g, 0, 0)),
        out_shape=jax.ShapeDtypeStruct((G, 1, D), jnp.float32),
    )(h, agg, W1, b1.reshape(1, D), W2, b2.reshape(1, D)).reshape(G, D)


def kernel(x, edge_index, ptr, W1_0, b1_0, W2_0, b2_0, W1_1, b1_1, W2_1, b2_1):
    src = edge_index[0].reshape(NCHUNKS, 1, C)
    dst = edge_index[1].reshape(NCHUNKS, 1, C)
    sc_agg = _get_sc_agg()
    xb = _to_bf16(x)
    agg0 = sc_agg(xb, src, dst)
    h1 = _mlp_mid(x, agg0, W1_0, b1_0, W2_0, b2_0)
    agg1 = sc_agg(h1, src, dst)
    return _mlp_pool(h1, agg1, W1_1, b1_1, W2_1, b2_1)


# speedup vs baseline: 1.2424x; 1.0322x over previous
import functools

import jax
import jax.numpy as jnp
from jax import lax
from jax.experimental import pallas as pl
from jax.experimental.pallas import tpu as pltpu
from jax.experimental.pallas import tpu_sc as plsc

N = 10000
E = 320000
D = 128
G = 10

NC = 2
NS = 16
C = 128
NCHUNKS = E // C
CPC = NCHUNKS // NC
MAIN = 78
NREM = CPC - NS * MAIN
NBUF = 6
GLA = 4
IGRP = 6
NGRP = MAIN // IGRP
SLOTS = 3
IROWS = SLOTS * IGRP
WB = 624
WREM = N - NS * WB
ZR = 48

def _sc_agg_body(h_hbm, src_hbm, dst_hbm, out_hbm, sidx, didx, rows, zbuf, acc,
                 isem, gsems, ssems):
    c = lax.axis_index("c")
    s = lax.axis_index("s")

    zv = jnp.zeros((32,), jnp.bfloat16)

    def _zrow(r, carry):
        for j in range(D // 32):
            zbuf[r, pl.ds(j * 32, 32)] = zv
        return carry

    lax.fori_loop(0, ZR, _zrow, 0)

    for k in range(WB // ZR):
        pltpu.sync_copy(zbuf, acc.at[pl.ds(s * WB + k * ZR, ZR)])

    @pl.when(s == NS - 1)
    def _zrem():
        pltpu.sync_copy(zbuf.at[pl.ds(0, WREM)], acc.at[pl.ds(NS * WB, WREM)])

    plsc.subcore_barrier()

    base = c * CPC + s * MAIN

    def _idx_load(grp, slot):
        off = pl.ds(slot * IGRP, IGRP)
        pltpu.async_copy(src_hbm.at[pl.ds(base + grp * IGRP, IGRP)],
                         sidx.at[off], isem)
        pltpu.async_copy(dst_hbm.at[pl.ds(base + grp * IGRP, IGRP)],
                         didx.at[off], isem)

    def _idx_wait():
        for _ in range(2):
            pltpu.make_async_copy(src_hbm.at[pl.ds(0, IGRP)],
                                  sidx.at[pl.ds(0, IGRP)], isem).wait()

    def _irow(j):
        return lax.rem(j // IGRP, SLOTS) * IGRP + lax.rem(j, IGRP)

    def _gather(j, b):
        pltpu.async_copy(h_hbm.at[sidx.at[_irow(j), 0]], rows.at[b], gsems[b])

    def _gather_wait(b):
        pltpu.make_async_copy(h_hbm.at[sidx.at[0, 0]], rows.at[b],
                              gsems[b]).wait()

    def _scatter(j, b):
        pltpu.async_copy(rows.at[b], acc.at[didx.at[_irow(j), 0]],
                         ssems[b], add=True)

    def _swait(b):
        pltpu.make_async_copy(rows.at[b], acc.at[didx.at[0, 0]],
                              ssems[b]).wait()

    _idx_load(0, 0)
    _idx_wait()
    _idx_load(1, 1)
    for b in range(GLA):
        _gather(b, b)

    _idx_wait()
    for k in range(IGRP):
        b = k % NBUF
        _gather_wait(b)
        _scatter(k, b)
        if k >= 2:
            _swait((b + GLA) % NBUF)
        if k == 2:
            _idx_load(2, 2)
        _gather(k + GLA, (b + GLA) % NBUF)

    def _outer(g, carry):
        @pl.when(g + 1 < NGRP)
        def _():
            _idx_wait()

        for k in range(IGRP):
            i = g * IGRP + k
            b = k % NBUF
            _gather_wait(b)
            _scatter(i, b)
            _swait((b + GLA) % NBUF)

            if k == 2:
                @pl.when(g + 2 < NGRP)
                def _():
                    _idx_load(g + 2, lax.rem(g - 1, SLOTS))

            gn = i + GLA

            @pl.when(gn < MAIN)
            def _():
                _gather(gn, (b + GLA) % NBUF)

        return carry

    lax.fori_loop(1, NGRP, _outer, 0)

    _swait((MAIN - 2) % NBUF)
    _swait((MAIN - 1) % NBUF)

    @pl.when(s < NREM)
    def _rem():
        chunk = c * CPC + NS * MAIN + s
        pltpu.sync_copy(src_hbm.at[chunk], sidx.at[0])
        pltpu.sync_copy(dst_hbm.at[chunk], didx.at[0])
        pltpu.async_copy(h_hbm.at[sidx.at[0, 0]], rows.at[0], gsems[0]).wait()
        pltpu.sync_copy(rows.at[0], acc.at[didx.at[0, 0]], add=True)

    plsc.subcore_barrier()

    pltpu.sync_copy(acc.at[pl.ds(s * WB, WB)],
                    out_hbm.at[c, pl.ds(s * WB, WB)])

    @pl.when(s == NS - 1)
    def _wrem():
        pltpu.sync_copy(acc.at[pl.ds(NS * WB, WREM)],
                        out_hbm.at[c, pl.ds(NS * WB, WREM)])


@functools.cache
def _get_sc_agg():
    mesh = plsc.VectorSubcoreMesh(core_axis_name="c", subcore_axis_name="s",
                                  num_cores=NC, num_subcores=NS)
    return pl.kernel(
        _sc_agg_body,
        out_type=jax.ShapeDtypeStruct((NC, N, D), jnp.bfloat16),
        mesh=mesh,
        compiler_params=pltpu.CompilerParams(use_tc_tiling_on_sc=False),
        scratch_types=[
            pltpu.VMEM((IROWS, 1, C), jnp.int32),
            pltpu.VMEM((IROWS, 1, C), jnp.int32),
            pltpu.VMEM((NBUF, C, D), jnp.bfloat16),
            pltpu.VMEM((ZR, D), jnp.bfloat16),
            pltpu.VMEM_SHARED((N, D), jnp.bfloat16),
            pltpu.SemaphoreType.DMA,
            [pltpu.SemaphoreType.DMA] * NBUF,
            [pltpu.SemaphoreType.DMA] * NBUF,
        ],
    )


RB = 2000
PB = N // G


def _cast_body(x_ref, o_ref):
    o_ref[...] = x_ref[...].astype(jnp.bfloat16)


def _to_bf16(x):
    return pl.pallas_call(
        _cast_body,
        grid=(N // RB,),
        in_specs=[pl.BlockSpec((RB, D), lambda g: (g, 0))],
        out_specs=pl.BlockSpec((RB, D), lambda g: (g, 0)),
        out_shape=jax.ShapeDtypeStruct((N, D), jnp.bfloat16),
    )(x)


def _mid_body(x_ref, a_ref, w1_ref, b1_ref, w2_ref, b2_ref, o_ref):
    z = (x_ref[...] + a_ref[0].astype(jnp.float32)
         + a_ref[1].astype(jnp.float32))
    z = jnp.dot(z, w1_ref[...], preferred_element_type=jnp.float32) + b1_ref[...]
    z = jnp.maximum(z, 0.0)
    z = jnp.dot(z, w2_ref[...], preferred_element_type=jnp.float32) + b2_ref[...]
    o_ref[...] = jnp.maximum(z, 0.0).astype(jnp.bfloat16)


def _mlp_mid(h, agg, W1, b1, W2, b2):
    return pl.pallas_call(
        _mid_body,
        grid=(N // RB,),
        in_specs=[
            pl.BlockSpec((RB, D), lambda g: (g, 0)),
            pl.BlockSpec((NC, RB, D), lambda g: (0, g, 0)),
            pl.BlockSpec((D, D), lambda g: (0, 0)),
            pl.BlockSpec((1, D), lambda g: (0, 0)),
            pl.BlockSpec((D, D), lambda g: (0, 0)),
            pl.BlockSpec((1, D), lambda g: (0, 0)),
        ],
        out_specs=pl.BlockSpec((RB, D), lambda g: (g, 0)),
        out_shape=jax.ShapeDtypeStruct((N, D), jnp.bfloat16),
    )(h, agg, W1, b1.reshape(1, D), W2, b2.reshape(1, D))


def _pool_body(x_ref, a_ref, w1_ref, b1_ref, w2_ref, b2_ref, o_ref):
    z = (x_ref[...].astype(jnp.float32) + a_ref[0].astype(jnp.float32)
         + a_ref[1].astype(jnp.float32))
    z = jnp.dot(z, w1_ref[...], preferred_element_type=jnp.float32) + b1_ref[...]
    z = jnp.maximum(z, 0.0)
    y = jnp.dot(z, w2_ref[...], preferred_element_type=jnp.float32) + b2_ref[...]
    o_ref[0] = jnp.sum(y, axis=0, keepdims=True) * (1.0 / PB)


def _mlp_pool(h, agg, W1, b1, W2, b2):
    return pl.pallas_call(
        _pool_body,
        grid=(G,),
        in_specs=[
            pl.BlockSpec((PB, D), lambda g: (g, 0)),
            pl.BlockSpec((NC, PB, D), lambda g: (0, g, 0)),
            pl.BlockSpec((D, D), lambda g: (0, 0)),
            pl.BlockSpec((1, D), lambda g: (0, 0)),
            pl.BlockSpec((D, D), lambda g: (0, 0)),
            pl.BlockSpec((1, D), lambda g: (0, 0)),
        ],
        out_specs=pl.BlockSpec((1, 1, D), lambda g: (g, 0, 0)),
        out_shape=jax.ShapeDtypeStruct((G, 1, D), jnp.float32),
    )(h, agg, W1, b1.reshape(1, D), W2, b2.reshape(1, D)).reshape(G, D)


def kernel(x, edge_index, ptr, W1_0, b1_0, W2_0, b2_0, W1_1, b1_1, W2_1, b2_1):
    src = edge_index[0].reshape(NCHUNKS, 1, C)
    dst = edge_index[1].reshape(NCHUNKS, 1, C)
    sc_agg = _get_sc_agg()
    xb = _to_bf16(x)
    agg0 = sc_agg(xb, src, dst)
    h1 = _mlp_mid(x, agg0, W1_0, b1_0, W2_0, b2_0)
    agg1 = sc_agg(h1, src, dst)
    return _mlp_pool(h1, agg1, W1_1, b1_1, W2_1, b2_1)


# speedup vs baseline: 1.2498x; 1.0060x over previous
import functools

import jax
import jax.numpy as jnp
from jax import lax
from jax.experimental import pallas as pl
from jax.experimental.pallas import tpu as pltpu
from jax.experimental.pallas import tpu_sc as plsc

N = 10000
E = 320000
D = 128
G = 10

NC = 2
NS = 16
C = 128
NCHUNKS = E // C
CPC = NCHUNKS // NC
MAIN = 78
NREM = CPC - NS * MAIN
NBUF = 6
GLA = 5
IGRP = 6
NGRP = MAIN // IGRP
SLOTS = 3
IROWS = SLOTS * IGRP
WB = 624
WREM = N - NS * WB
ZR = 48

def _sc_agg_body(h_hbm, src_hbm, dst_hbm, out_hbm, sidx, didx, rows, zbuf, acc,
                 isem, gsems, ssems):
    c = lax.axis_index("c")
    s = lax.axis_index("s")

    zv = jnp.zeros((32,), jnp.bfloat16)

    def _zrow(r, carry):
        for j in range(D // 32):
            zbuf[r, pl.ds(j * 32, 32)] = zv
        return carry

    lax.fori_loop(0, ZR, _zrow, 0)

    for k in range(WB // ZR):
        pltpu.sync_copy(zbuf, acc.at[pl.ds(s * WB + k * ZR, ZR)])

    @pl.when(s == NS - 1)
    def _zrem():
        pltpu.sync_copy(zbuf.at[pl.ds(0, WREM)], acc.at[pl.ds(NS * WB, WREM)])

    plsc.subcore_barrier()

    base = c * CPC + s * MAIN

    def _idx_load(grp, slot):
        off = pl.ds(slot * IGRP, IGRP)
        pltpu.async_copy(src_hbm.at[pl.ds(base + grp * IGRP, IGRP)],
                         sidx.at[off], isem)
        pltpu.async_copy(dst_hbm.at[pl.ds(base + grp * IGRP, IGRP)],
                         didx.at[off], isem)

    def _idx_wait():
        for _ in range(2):
            pltpu.make_async_copy(src_hbm.at[pl.ds(0, IGRP)],
                                  sidx.at[pl.ds(0, IGRP)], isem).wait()

    def _irow(j):
        return lax.rem(j // IGRP, SLOTS) * IGRP + lax.rem(j, IGRP)

    def _gather(j, b):
        pltpu.async_copy(h_hbm.at[sidx.at[_irow(j), 0]], rows.at[b], gsems[b])

    def _gather_wait(b):
        pltpu.make_async_copy(h_hbm.at[sidx.at[0, 0]], rows.at[b],
                              gsems[b]).wait()

    def _scatter(j, b):
        pltpu.async_copy(rows.at[b], acc.at[didx.at[_irow(j), 0]],
                         ssems[b], add=True)

    def _swait(b):
        pltpu.make_async_copy(rows.at[b], acc.at[didx.at[0, 0]],
                              ssems[b]).wait()

    _idx_load(0, 0)
    _idx_wait()
    _idx_load(1, 1)
    for b in range(GLA):
        _gather(b, b)

    _idx_wait()
    for k in range(IGRP):
        b = k % NBUF
        _gather_wait(b)
        _scatter(k, b)
        if k >= 1:
            _swait((b + GLA) % NBUF)
        if k == 2:
            _idx_load(2, 2)
        _gather(k + GLA, (b + GLA) % NBUF)

    def _outer(g, carry):
        @pl.when(g + 1 < NGRP)
        def _():
            _idx_wait()

        for k in range(IGRP):
            i = g * IGRP + k
            b = k % NBUF
            _gather_wait(b)
            _scatter(i, b)
            _swait((b + GLA) % NBUF)

            if k == 2:
                @pl.when(g + 2 < NGRP)
                def _():
                    _idx_load(g + 2, lax.rem(g - 1, SLOTS))

            gn = i + GLA

            @pl.when(gn < MAIN)
            def _():
                _gather(gn, (b + GLA) % NBUF)

        return carry

    lax.fori_loop(1, NGRP, _outer, 0)

    _swait((MAIN - 1) % NBUF)

    @pl.when(s < NREM)
    def _rem():
        chunk = c * CPC + NS * MAIN + s
        pltpu.sync_copy(src_hbm.at[chunk], sidx.at[0])
        pltpu.sync_copy(dst_hbm.at[chunk], didx.at[0])
        pltpu.async_copy(h_hbm.at[sidx.at[0, 0]], rows.at[0], gsems[0]).wait()
        pltpu.sync_copy(rows.at[0], acc.at[didx.at[0, 0]], add=True)

    plsc.subcore_barrier()

    pltpu.sync_copy(acc.at[pl.ds(s * WB, WB)],
                    out_hbm.at[c, pl.ds(s * WB, WB)])

    @pl.when(s == NS - 1)
    def _wrem():
        pltpu.sync_copy(acc.at[pl.ds(NS * WB, WREM)],
                        out_hbm.at[c, pl.ds(NS * WB, WREM)])


@functools.cache
def _get_sc_agg():
    mesh = plsc.VectorSubcoreMesh(core_axis_name="c", subcore_axis_name="s",
                                  num_cores=NC, num_subcores=NS)
    return pl.kernel(
        _sc_agg_body,
        out_type=jax.ShapeDtypeStruct((NC, N, D), jnp.bfloat16),
        mesh=mesh,
        compiler_params=pltpu.CompilerParams(use_tc_tiling_on_sc=False),
        scratch_types=[
            pltpu.VMEM((IROWS, 1, C), jnp.int32),
            pltpu.VMEM((IROWS, 1, C), jnp.int32),
            pltpu.VMEM((NBUF, C, D), jnp.bfloat16),
            pltpu.VMEM((ZR, D), jnp.bfloat16),
            pltpu.VMEM_SHARED((N, D), jnp.bfloat16),
            pltpu.SemaphoreType.DMA,
            [pltpu.SemaphoreType.DMA] * NBUF,
            [pltpu.SemaphoreType.DMA] * NBUF,
        ],
    )


RB = 2000
PB = N // G


def _cast_body(x_ref, o_ref):
    o_ref[...] = x_ref[...].astype(jnp.bfloat16)


def _to_bf16(x):
    return pl.pallas_call(
        _cast_body,
        grid=(N // RB,),
        in_specs=[pl.BlockSpec((RB, D), lambda g: (g, 0))],
        out_specs=pl.BlockSpec((RB, D), lambda g: (g, 0)),
        out_shape=jax.ShapeDtypeStruct((N, D), jnp.bfloat16),
    )(x)


def _mid_body(x_ref, a_ref, w1_ref, b1_ref, w2_ref, b2_ref, o_ref):
    z = (x_ref[...] + a_ref[0].astype(jnp.float32)
         + a_ref[1].astype(jnp.float32))
    z = jnp.dot(z, w1_ref[...], preferred_element_type=jnp.float32) + b1_ref[...]
    z = jnp.maximum(z, 0.0)
    z = jnp.dot(z, w2_ref[...], preferred_element_type=jnp.float32) + b2_ref[...]
    o_ref[...] = jnp.maximum(z, 0.0).astype(jnp.bfloat16)


def _mlp_mid(h, agg, W1, b1, W2, b2):
    return pl.pallas_call(
        _mid_body,
        grid=(N // RB,),
        in_specs=[
            pl.BlockSpec((RB, D), lambda g: (g, 0)),
            pl.BlockSpec((NC, RB, D), lambda g: (0, g, 0)),
            pl.BlockSpec((D, D), lambda g: (0, 0)),
            pl.BlockSpec((1, D), lambda g: (0, 0)),
            pl.BlockSpec((D, D), lambda g: (0, 0)),
            pl.BlockSpec((1, D), lambda g: (0, 0)),
        ],
        out_specs=pl.BlockSpec((RB, D), lambda g: (g, 0)),
        out_shape=jax.ShapeDtypeStruct((N, D), jnp.bfloat16),
    )(h, agg, W1, b1.reshape(1, D), W2, b2.reshape(1, D))


def _pool_body(x_ref, a_ref, w1_ref, b1_ref, w2_ref, b2_ref, o_ref):
    z = (x_ref[...].astype(jnp.float32) + a_ref[0].astype(jnp.float32)
         + a_ref[1].astype(jnp.float32))
    z = jnp.dot(z, w1_ref[...], preferred_element_type=jnp.float32) + b1_ref[...]
    z = jnp.maximum(z, 0.0)
    y = jnp.dot(z, w2_ref[...], preferred_element_type=jnp.float32) + b2_ref[...]
    o_ref[0] = jnp.sum(y, axis=0, keepdims=True) * (1.0 / PB)


def _mlp_pool(h, agg, W1, b1, W2, b2):
    return pl.pallas_call(
        _pool_body,
        grid=(G,),
        in_specs=[
            pl.BlockSpec((PB, D), lambda g: (g, 0)),
            pl.BlockSpec((NC, PB, D), lambda g: (0, g, 0)),
            pl.BlockSpec((D, D), lambda g: (0, 0)),
            pl.BlockSpec((1, D), lambda g: (0, 0)),
            pl.BlockSpec((D, D), lambda g: (0, 0)),
            pl.BlockSpec((1, D), lambda g: (0, 0)),
        ],
        out_specs=pl.BlockSpec((1, 1, D), lambda g: (g, 0, 0)),
        out_shape=jax.ShapeDtypeStruct((G, 1, D), jnp.float32),
    )(h, agg, W1, b1.reshape(1, D), W2, b2.reshape(1, D)).reshape(G, D)


def kernel(x, edge_index, ptr, W1_0, b1_0, W2_0, b2_0, W1_1, b1_1, W2_1, b2_1):
    src = edge_index[0].reshape(NCHUNKS, 1, C)
    dst = edge_index[1].reshape(NCHUNKS, 1, C)
    sc_agg = _get_sc_agg()
    xb = _to_bf16(x)
    agg0 = sc_agg(xb, src, dst)
    h1 = _mlp_mid(x, agg0, W1_0, b1_0, W2_0, b2_0)
    agg1 = sc_agg(h1, src, dst)
    return _mlp_pool(h1, agg1, W1_1, b1_1, W2_1, b2_1)
